# Initial kernel scaffold; baseline (speedup 1.0000x reference)
import jax
import jax.numpy as jnp
from jax.experimental import pallas as pl


def kernel(x, atom_scalar, edge_index, dist, vec_norm, rbf, emb, ae0, ae1, ae2, ae3, ae4, ae5, ae6, ae7, ae8, W1, b1, W2, b2, W3, b3, emb2_W, emb2_b, ln_g, ln_b, Wt0, Wt1, Wt2, Ws1_W, Ws1_b, Ws2_W, Ws2_b):
    raise NotImplementedError("write your pallas kernel here")


# speedup vs baseline: 50.3347x; 50.3347x over previous
import functools

import jax
import jax.numpy as jnp
from jax import lax
from jax.experimental import pallas as pl
from jax.experimental.pallas import tpu as pltpu
from jax.experimental.pallas import tpu_sc as plsc

N = 10000
E = 160000
H = 128
R = 64
TAB = 256
NG = 9
EROWS = E // 128
F32 = jnp.float32


def _embed_body(idx_ref, tab_ref, wl_ref, wr_ref, b_ref, cs_ref, cd_ref):
    idx = idx_ref[...]
    cols = lax.broadcasted_iota(jnp.int32, (1, TAB), 1)
    oh = jnp.zeros((idx.shape[0], TAB), F32)
    for i in range(10):
        oh = oh + (idx[:, i:i + 1] == cols).astype(F32)
    z = jnp.dot(oh, tab_ref[...], preferred_element_type=F32)
    cs_ref[...] = lax.dot_general(z, wl_ref[...], (((1,), (1,)), ((), ())),
                                  preferred_element_type=F32)
    cd_ref[...] = lax.dot_general(z, wr_ref[...], (((1,), (1,)), ((), ())),
                                  preferred_element_type=F32) + b_ref[...]


def _embed(idx10, tab, wl, wr, b2):
    blk = 1000
    return pl.pallas_call(
        _embed_body,
        grid=(N // blk,),
        in_specs=[
            pl.BlockSpec((blk, 10), lambda i: (i, 0)),
            pl.BlockSpec((TAB, H), lambda i: (0, 0)),
            pl.BlockSpec((H, H), lambda i: (0, 0)),
            pl.BlockSpec((H, H), lambda i: (0, 0)),
            pl.BlockSpec((1, H), lambda i: (0, 0)),
        ],
        out_specs=[
            pl.BlockSpec((blk, H), lambda i: (i, 0)),
            pl.BlockSpec((blk, H), lambda i: (i, 0)),
        ],
        out_shape=[
            jax.ShapeDtypeStruct((N, H), F32),
            jax.ShapeDtypeStruct((N, H), F32),
        ],
    )(idx10, tab, wl, wr, b2)


def _gather_body(src_hbm, dst_hbm, cs_hbm, cd_hbm, csg_hbm, cdg_hbm,
                 idxs, idxd, ba0, ba1, bb0, bb1, sa0, sa1, sb0, sb1):
    cid = lax.axis_index("c")
    sid = lax.axis_index("s")
    wid = sid * 2 + cid
    r0 = wid * 40
    nr = jnp.where(wid < 31, 40, 10)

    @pl.when(wid < 31)
    def _():
        pltpu.sync_copy(src_hbm.at[pl.ds(r0, 40), :], idxs)
        pltpu.sync_copy(dst_hbm.at[pl.ds(r0, 40), :], idxd)

    @pl.when(wid == 31)
    def _():
        pltpu.sync_copy(src_hbm.at[pl.ds(1240, 10), :],
                        idxs.at[pl.ds(0, 10), :])
        pltpu.sync_copy(dst_hbm.at[pl.ds(1240, 10), :],
                        idxd.at[pl.ds(0, 10), :])

    def start(r, ba, bb, sa, sb):
        pltpu.make_async_copy(cs_hbm.at[idxs.at[r]], ba, sa).start()
        pltpu.make_async_copy(cd_hbm.at[idxd.at[r]], bb, sb).start()

    def finish(r, ba, bb, sa, sb):
        pltpu.make_async_copy(cs_hbm.at[idxs.at[r]], ba, sa).wait()
        pltpu.make_async_copy(cd_hbm.at[idxd.at[r]], bb, sb).wait()
        e0 = (r0 + r) * 128
        pltpu.sync_copy(ba, csg_hbm.at[pl.ds(e0, 128), :])
        pltpu.sync_copy(bb, cdg_hbm.at[pl.ds(e0, 128), :])

    start(0, ba0, bb0, sa0, sb0)

    def pair(k, _):
        start(2 * k + 1, ba1, bb1, sa1, sb1)
        finish(2 * k, ba0, bb0, sa0, sb0)
        start(2 * k + 2, ba0, bb0, sa0, sb0)
        finish(2 * k + 1, ba1, bb1, sa1, sb1)
        return _

    lax.fori_loop(0, nr // 2 - 1, pair, 0)
    start(nr - 1, ba1, bb1, sa1, sb1)
    finish(nr - 2, ba0, bb0, sa0, sb0)
    finish(nr - 1, ba1, bb1, sa1, sb1)


def _sc_gather(cs, cd, src2d, dst2d):
    f = pl.kernel(
        _gather_body,
        out_type=[
            jax.ShapeDtypeStruct((E, H), F32),
            jax.ShapeDtypeStruct((E, H), F32),
        ],
        mesh=plsc.VectorSubcoreMesh(core_axis_name="c", subcore_axis_name="s",
                                    num_cores=2, num_subcores=16),
        scratch_types=[
            pltpu.VMEM((40, 128), jnp.int32),
            pltpu.VMEM((40, 128), jnp.int32),
            pltpu.VMEM((128, H), F32),
            pltpu.VMEM((128, H), F32),
            pltpu.VMEM((128, H), F32),
            pltpu.VMEM((128, H), F32),
            pltpu.SemaphoreType.DMA,
            pltpu.SemaphoreType.DMA,
            pltpu.SemaphoreType.DMA,
            pltpu.SemaphoreType.DMA,
        ],
    )
    return f(src2d, dst2d, cs, cd)


def _edge_body(rbf_ref, csg_ref, cdg_ref, v_ref, w1_ref, b1_ref, w2_ref,
               b2_ref, w3_ref, b3_ref, msg_ref):
    c = csg_ref[...] + cdg_ref[...]
    rbf = rbf_ref[...]
    dn = (((1,), (1,)), ((), ()))
    u1 = (lax.dot_general(rbf, w1_ref[...], dn, preferred_element_type=F32)
          + b1_ref[...]) * c
    u2 = (lax.dot_general(rbf, w2_ref[...], dn, preferred_element_type=F32)
          + b2_ref[...]) * c
    u3 = (lax.dot_general(rbf, w3_ref[...], dn, preferred_element_type=F32)
          + b3_ref[...]) * c
    v = v_ref[...]
    v0, v1, v2 = v[:, 0:1], v[:, 1:2], v[:, 2:3]
    msg_ref[0] = u1
    msg_ref[1] = u2 * v0
    msg_ref[2] = u2 * v1
    msg_ref[3] = u2 * v2
    msg_ref[4] = u3 * (v0 * v0 - v2 * v2)
    msg_ref[5] = u3 * (v1 * v1 - v2 * v2)
    msg_ref[6] = u3 * (v0 * v1)
    msg_ref[7] = u3 * (v0 * v2)
    msg_ref[8] = u3 * (v1 * v2)


def _edge_msg(rbf, csg, cdg, vec_norm, w1, b1, w2, b2, w3, b3):
    blk = 640
    wspec = pl.BlockSpec((H, R), lambda i: (0, 0))
    bspec = pl.BlockSpec((1, H), lambda i: (0, 0))
    return pl.pallas_call(
        _edge_body,
        grid=(E // blk,),
        in_specs=[
            pl.BlockSpec((blk, R), lambda i: (i, 0)),
            pl.BlockSpec((blk, H), lambda i: (i, 0)),
            pl.BlockSpec((blk, H), lambda i: (i, 0)),
            pl.BlockSpec((blk, 3), lambda i: (i, 0)),
            wspec, bspec, wspec, bspec, wspec, bspec,
        ],
        out_specs=pl.BlockSpec((NG, blk, H), lambda i: (0, i, 0)),
        out_shape=jax.ShapeDtypeStruct((NG, E, H), F32),
    )(rbf, csg, cdg, vec_norm, w1, b1, w2, b2, w3, b3)


def _scatter_body(dst_hbm, msg_hbm, p_hbm,
                  acc, idx, zbuf, b0, b1, s0, s1):
    cid = lax.axis_index("c")
    sid = lax.axis_index("s")
    r0 = sid * 80
    nb = jnp.where(sid < 15, 80, 50)

    z16 = jnp.zeros((16,), F32)

    def zrow(r, _):
        for cc in range(8):
            zbuf[r, pl.ds(cc * 16, 16)] = z16
        return _

    lax.fori_loop(0, 16, zrow, 0)

    @pl.when(sid < 15)
    def _():
        pltpu.sync_copy(dst_hbm.at[pl.ds(r0, 80), :], idx)

    @pl.when(sid == 15)
    def _():
        pltpu.sync_copy(dst_hbm.at[pl.ds(1200, 50), :],
                        idx.at[pl.ds(0, 50), :])

    def start(g, r, buf, sem):
        e0 = (r0 + r) * 128
        pltpu.make_async_copy(msg_hbm.at[g, pl.ds(e0, 128), :], buf, sem).start()

    def finish(g, r, buf, sem):
        e0 = (r0 + r) * 128
        pltpu.make_async_copy(msg_hbm.at[g, pl.ds(e0, 128), :], buf, sem).wait()
        pltpu.sync_copy(buf, acc.at[idx.at[r]], add=True)

    for g in range(NG):
        owner = 0 if g < 5 else 1

        @pl.when(cid == owner)
        def _(g=g):
            def zslab(j, _):
                pltpu.sync_copy(zbuf, acc.at[pl.ds(sid * 624 + j * 16, 16), :])
                return _

            lax.fori_loop(0, 39, zslab, 0)

            @pl.when(sid == 15)
            def _():
                pltpu.sync_copy(zbuf, acc.at[pl.ds(9984, 16), :])

            plsc.subcore_barrier()

            start(g, 0, b0, s0)

            def pair(k, _):
                start(g, 2 * k + 1, b1, s1)
                finish(g, 2 * k, b0, s0)
                start(g, 2 * k + 2, b0, s0)
                finish(g, 2 * k + 1, b1, s1)
                return _

            lax.fori_loop(0, nb // 2 - 1, pair, 0)
            start(g, nb - 1, b1, s1)
            finish(g, nb - 2, b0, s0)
            finish(g, nb - 1, b1, s1)

            plsc.subcore_barrier()
            pltpu.sync_copy(acc.at[pl.ds(sid * 624, 624), :],
                            p_hbm.at[g, pl.ds(sid * 624, 624), :])

            @pl.when(sid == 15)
            def _():
                pltpu.sync_copy(acc.at[pl.ds(9984, 16), :],
                                p_hbm.at[g, pl.ds(9984, 16), :])

            plsc.subcore_barrier()


def _sc_scatter(msg, dst2d):
    f = pl.kernel(
        _scatter_body,
        out_type=jax.ShapeDtypeStruct((NG, N, H), F32),
        mesh=plsc.VectorSubcoreMesh(core_axis_name="c", subcore_axis_name="s",
                                    num_cores=2, num_subcores=16),
        scratch_types=[
            pltpu.VMEM_SHARED((N, H), F32),
            pltpu.VMEM((80, 128), jnp.int32),
            pltpu.VMEM((16, H), F32),
            pltpu.VMEM((128, H), F32),
            pltpu.VMEM((128, H), F32),
            pltpu.SemaphoreType.DMA,
            pltpu.SemaphoreType.DMA,
        ],
    )
    return f(dst2d, msg)


def _post_body(p_ref, lng_ref, lnb_ref, ws1_ref, ws1b_ref,
               w20_ref, b20_ref, w21_ref, b21_ref, w22_ref, b22_ref,
               wt0_ref, wt1_ref, wt2_ref, out_ref):
    p0 = p_ref[0]
    pa0, pa1, pa2 = p_ref[1], p_ref[2], p_ref[3]
    pt0, pt1 = p_ref[4], p_ref[5]
    p01, p02, p12 = p_ref[6], p_ref[7], p_ref[8]

    s00 = (2.0 * pt0 - pt1) / 3.0
    s11 = (2.0 * pt1 - pt0) / 3.0
    s22 = -(pt0 + pt1) / 3.0
    fro = (3.0 * p0 * p0
           + 2.0 * (pa0 * pa0 + pa1 * pa1 + pa2 * pa2)
           + s00 * s00 + s11 * s11 + s22 * s22
           + 2.0 * (p01 * p01 + p02 * p02 + p12 * p12))

    mu = jnp.mean(fro, axis=-1, keepdims=True)
    var = jnp.mean((fro - mu) ** 2, axis=-1, keepdims=True)
    y = lng_ref[...] * (fro - mu) * lax.rsqrt(var + 1e-5) + lnb_ref[...]

    dn = (((1,), (1,)), ((), ()))

    def silu(t):
        return t * lax.logistic(t)

    h1 = silu(lax.dot_general(y, ws1_ref[...], dn, preferred_element_type=F32)
              + ws1b_ref[...])
    n0 = silu(lax.dot_general(h1, w20_ref[...], dn, preferred_element_type=F32)
              + b20_ref[...])
    n1 = silu(lax.dot_general(h1, w21_ref[...], dn, preferred_element_type=F32)
              + b21_ref[...])
    n2 = silu(lax.dot_general(h1, w22_ref[...], dn, preferred_element_type=F32)
              + b22_ref[...])

    a = lax.dot_general(p0, wt0_ref[...], dn, preferred_element_type=F32) * n0
    wt1 = wt1_ref[...]
    b0 = lax.dot_general(pa0, wt1, dn, preferred_element_type=F32) * n1
    b1 = lax.dot_general(pa1, wt1, dn, preferred_element_type=F32) * n1
    b2 = lax.dot_general(pa2, wt1, dn, preferred_element_type=F32) * n1
    wt2 = wt2_ref[...]
    t0 = lax.dot_general(pt0, wt2, dn, preferred_element_type=F32) * n2
    t1 = lax.dot_general(pt1, wt2, dn, preferred_element_type=F32) * n2
    q01 = lax.dot_general(p01, wt2, dn, preferred_element_type=F32) * n2
    q02 = lax.dot_general(p02, wt2, dn, preferred_element_type=F32) * n2
    q12 = lax.dot_general(p12, wt2, dn, preferred_element_type=F32) * n2

    o00 = (2.0 * t0 - t1) / 3.0
    o11 = (2.0 * t1 - t0) / 3.0
    o22 = -(t0 + t1) / 3.0
    out_ref[0] = a + o00
    out_ref[1] = -b2 + q01
    out_ref[2] = b1 + q02
    out_ref[3] = b2 + q01
    out_ref[4] = a + o11
    out_ref[5] = -b0 + q12
    out_ref[6] = -b1 + q02
    out_ref[7] = b0 + q12
    out_ref[8] = a + o22


def _node_post(p, ln_g, ln_b, ws1, ws1b, w20, b20, w21, b21, w22, b22,
               wt0, wt1, wt2):
    blk = 1000
    hh = pl.BlockSpec((H, H), lambda i: (0, 0))
    bias = pl.BlockSpec((1, H), lambda i: (0, 0))
    return pl.pallas_call(
        _post_body,
        grid=(N // blk,),
        in_specs=[
            pl.BlockSpec((NG, blk, H), lambda i: (0, i, 0)),
            bias, bias,
            pl.BlockSpec((2 * H, H), lambda i: (0, 0)),
            pl.BlockSpec((1, 2 * H), lambda i: (0, 0)),
            pl.BlockSpec((H, 2 * H), lambda i: (0, 0)), bias,
            pl.BlockSpec((H, 2 * H), lambda i: (0, 0)), bias,
            pl.BlockSpec((H, 2 * H), lambda i: (0, 0)), bias,
            hh, hh, hh,
        ],
        out_specs=pl.BlockSpec((NG, blk, H), lambda i: (0, i, 0)),
        out_shape=jax.ShapeDtypeStruct((NG, N, H), F32),
    )(p, ln_g, ln_b, ws1, ws1b, w20, b20, w21, b21, w22, b22, wt0, wt1, wt2)


def kernel(x, atom_scalar, edge_index, dist, vec_norm, rbf, emb,
           ae0, ae1, ae2, ae3, ae4, ae5, ae6, ae7, ae8,
           W1, b1, W2, b2, W3, b3, emb2_W, emb2_b, ln_g, ln_b,
           Wt0, Wt1, Wt2, Ws1_W, Ws1_b, Ws2_W, Ws2_b):
    del dist

    aes = [ae0, ae1, ae2, ae3, ae4, ae5, ae6, ae7, ae8]
    dims = [a.shape[0] for a in aes]
    tab = jnp.concatenate(
        [emb] + aes + [jnp.zeros((TAB - 128 - sum(dims), H), F32)], axis=0)
    offs, o = [], 128
    for d in dims:
        offs.append(o)
        o += d
    idx10 = jnp.concatenate(
        [x[:, None]] + [atom_scalar[:, i:i + 1] + offs[i] for i in range(9)],
        axis=1).astype(jnp.int32)

    wl = emb2_W[:, :H]
    wr = emb2_W[:, H:]
    src2d = edge_index[0].astype(jnp.int32).reshape(EROWS, 128)
    dst2d = edge_index[1].astype(jnp.int32).reshape(EROWS, 128)

    cs, cd = _embed(idx10, tab, wl, wr, emb2_b.reshape(1, H))
    csg, cdg = _sc_gather(cs, cd, src2d, dst2d)
    msg = _edge_msg(rbf, csg, cdg, vec_norm,
                    W1, b1.reshape(1, H), W2, b2.reshape(1, H),
                    W3, b3.reshape(1, H))
    p = _sc_scatter(msg, dst2d)
    out9 = _node_post(p, ln_g.reshape(1, H), ln_b.reshape(1, H),
                      Ws1_W, Ws1_b.reshape(1, 2 * H),
                      Ws2_W[0::3], Ws2_b[0::3].reshape(1, H),
                      Ws2_W[1::3], Ws2_b[1::3].reshape(1, H),
                      Ws2_W[2::3], Ws2_b[2::3].reshape(1, H),
                      Wt0, Wt1, Wt2)
    return jnp.transpose(out9, (1, 2, 0)).reshape(N, H, 3, 3)
